# baseline (speedup 1.0000x reference)
import functools

import jax
import jax.numpy as jnp
from jax import lax
from jax.experimental import pallas as pl
from jax.experimental.pallas import tpu as pltpu
from jax.experimental.pallas import tpu_sc as plsc

B, N, D = 4, 2048, 2048
NUM_CORES = 2
NUM_SUBCORES = 16
NW = NUM_CORES * NUM_SUBCORES
ROWS_PER_W = N // NW
CHUNK = 16
NCHUNK = ROWS_PER_W // CHUNK
NBUF = 3

_mesh = plsc.VectorSubcoreMesh(core_axis_name="c", subcore_axis_name="s")


@functools.partial(
    pl.kernel,
    mesh=_mesh,
    out_type=jax.ShapeDtypeStruct((N, D), jnp.float32),
    scratch_types=(
        [pltpu.VMEM((CHUNK, D), jnp.float32) for _ in range(NBUF)]
        + [pltpu.SemaphoreType.DMA for _ in range(2 * NBUF)]
    ),
)
def _sc_batch0(w_hbm, out_hbm, *scratch):
    bufs = scratch[:NBUF]
    rsem = scratch[NBUF:2 * NBUF]
    wsem = scratch[2 * NBUF:]
    wid = lax.axis_index("s") * NUM_CORES + lax.axis_index("c")
    base = wid * ROWS_PER_W

    reads = [None] * NCHUNK
    writes = [None] * NCHUNK
    drained = set()

    def start_read(i):
        r0 = base + i * CHUNK
        reads[i] = pltpu.async_copy(
            w_hbm.at[pl.ds(r0, CHUNK), :], bufs[i % NBUF], rsem[i % NBUF])

    for i in range(min(NBUF, NCHUNK)):
        start_read(i)
    for i in range(NCHUNK):
        reads[i].wait()
        r0 = base + i * CHUNK
        writes[i] = [pltpu.async_copy(
            bufs[i % NBUF], out_hbm.at[pl.ds(r0, CHUNK), :], wsem[i % NBUF])]
        j = i + NBUF - 1
        if NBUF <= j < NCHUNK:
            for h in writes[j - NBUF]:
                h.wait()
            drained.add(j - NBUF)
            start_read(j)
    for i in range(NCHUNK):
        if i not in drained:
            for h in writes[i]:
                h.wait()


BLK = 128
GRID = N // BLK


def _tc_body(w_ref, o_ref):
    o_ref[...] = jnp.broadcast_to(w_ref[...][None, :, :], (B - 1, BLK, D))


def kernel(x, embed_weight):
    b, n = x.shape
    sc0 = _sc_batch0(embed_weight)
    tc123 = pl.pallas_call(
        _tc_body,
        grid=(GRID,),
        in_specs=[pl.BlockSpec((BLK, D), lambda i: (i, 0))],
        out_specs=pl.BlockSpec((B - 1, BLK, D), lambda i: (0, i, 0)),
        out_shape=jax.ShapeDtypeStruct((B - 1, N, D), jnp.float32),
    )(embed_weight)
    return jnp.concatenate([sc0[None, :, :], tc123], axis=0)


# speedup vs baseline: 1.9264x; 1.9264x over previous
import functools

import jax
import jax.numpy as jnp
from jax import lax
from jax.experimental import pallas as pl
from jax.experimental.pallas import tpu as pltpu
from jax.experimental.pallas import tpu_sc as plsc

B, N, D = 4, 2048, 2048
NUM_CORES = 2
NUM_SUBCORES = 16
NW = NUM_CORES * NUM_SUBCORES
ROWS_PER_W = N // NW
CHUNK = 16
NCHUNK = ROWS_PER_W // CHUNK

_mesh = plsc.VectorSubcoreMesh(core_axis_name="c", subcore_axis_name="s")


@functools.partial(
    pl.kernel,
    mesh=_mesh,
    out_type=jax.ShapeDtypeStruct((B * N, D), jnp.float32),
    scratch_types=[
        pltpu.VMEM((CHUNK, D), jnp.float32),
        pltpu.SemaphoreType.DMA,
    ],
)
def _bcast_copy(w_hbm, out_hbm, buf, sem):
    wid = lax.axis_index("s") * NUM_CORES + lax.axis_index("c")
    base = wid * ROWS_PER_W

    def chunk_body(i, carry):
        r0 = base + i * CHUNK
        pltpu.sync_copy(w_hbm.at[pl.ds(r0, CHUNK), :], buf)

        def batch_body(b, c2):
            pltpu.sync_copy(buf, out_hbm.at[pl.ds(b * N + r0, CHUNK), :])
            return c2

        return lax.fori_loop(0, B, batch_body, carry)

    lax.fori_loop(0, NCHUNK, chunk_body, 0)


def kernel(x, embed_weight):
    b, n = x.shape
    out = _bcast_copy(embed_weight)
    return out.reshape(b, n, D)


# speedup vs baseline: 2.0035x; 1.0400x over previous
import functools

import jax
import jax.numpy as jnp
from jax import lax
from jax.experimental import pallas as pl
from jax.experimental.pallas import tpu as pltpu
from jax.experimental.pallas import tpu_sc as plsc

B, N, D = 4, 2048, 2048
NUM_CORES = 2
NUM_SUBCORES = 16
ROWS_PER_SC = N // NUM_CORES
SPM = 256
T_ROWS = (ROWS_PER_SC - SPM) // NUM_SUBCORES
CHUNK = 16
NCHUNK = T_ROWS // CHUNK

_mesh = plsc.VectorSubcoreMesh(core_axis_name="c", subcore_axis_name="s")


@functools.partial(
    pl.kernel,
    mesh=_mesh,
    out_type=jax.ShapeDtypeStruct((B * N, D), jnp.float32),
    scratch_types=(
        [pltpu.VMEM((CHUNK, D), jnp.float32) for _ in range(NCHUNK)]
        + [pltpu.SemaphoreType.DMA for _ in range(2 * NCHUNK)]
        + [pltpu.VMEM_SHARED((SPM, D), jnp.float32), pltpu.SemaphoreType.DMA]
    ),
)
def _bcast_copy(w_hbm, out_hbm, *scratch):
    bufs = scratch[:NCHUNK]
    rsem = scratch[NCHUNK:2 * NCHUNK]
    wsem = scratch[2 * NCHUNK:3 * NCHUNK]
    spm = scratch[3 * NCHUNK]
    ssem = scratch[3 * NCHUNK + 1]
    c = lax.axis_index("c")
    s = lax.axis_index("s")
    sc_base = c * ROWS_PER_SC

    @pl.when(s == 0)
    def _spmem_start():
        pltpu.sync_copy(w_hbm.at[pl.ds(sc_base, SPM), :], spm)
        for b in range(B):
            pltpu.async_copy(
                spm, out_hbm.at[pl.ds(b * N + sc_base, SPM), :], ssem)

    base = sc_base + SPM + s * T_ROWS
    reads = []
    for i in range(NCHUNK):
        reads.append(pltpu.async_copy(
            w_hbm.at[pl.ds(base + i * CHUNK, CHUNK), :], bufs[i], rsem[i]))
    writes = []
    for i in range(NCHUNK):
        reads[i].wait()
        r0 = base + i * CHUNK
        writes += [
            pltpu.async_copy(
                bufs[i], out_hbm.at[pl.ds(b * N + r0, CHUNK), :], wsem[i])
            for b in range(B)
        ]
    for h in writes:
        h.wait()

    @pl.when(s == 0)
    def _spmem_drain():
        for b in range(B):
            pltpu.make_async_copy(
                spm, out_hbm.at[pl.ds(b * N + sc_base, SPM), :], ssem).wait()


def kernel(x, embed_weight):
    b, n = x.shape
    out = _bcast_copy(embed_weight)
    return out.reshape(b, n, D)


# speedup vs baseline: 2.0088x; 1.0027x over previous
import functools

import jax
import jax.numpy as jnp
from jax import lax
from jax.experimental import pallas as pl
from jax.experimental.pallas import tpu as pltpu
from jax.experimental.pallas import tpu_sc as plsc

B, N, D = 4, 2048, 2048
NUM_CORES = 2
NUM_SUBCORES = 16
NW = NUM_CORES * NUM_SUBCORES
ROWS_PER_W = N // NW
CHUNK = 16
NCHUNK = ROWS_PER_W // CHUNK
NBUF = 3

_mesh = plsc.VectorSubcoreMesh(core_axis_name="c", subcore_axis_name="s")


@functools.partial(
    pl.kernel,
    mesh=_mesh,
    out_type=jax.ShapeDtypeStruct((B * N, D), jnp.float32),
    scratch_types=(
        [pltpu.VMEM((CHUNK, D), jnp.float32) for _ in range(NBUF)]
        + [pltpu.SemaphoreType.DMA for _ in range(2 * NBUF)]
    ),
)
def _bcast_copy(w_hbm, out_hbm, *scratch):
    bufs = scratch[:NBUF]
    rsem = scratch[NBUF:2 * NBUF]
    wsem = scratch[2 * NBUF:]
    wid = lax.axis_index("s") * NUM_CORES + lax.axis_index("c")
    base = wid * ROWS_PER_W

    reads = [None] * NCHUNK
    writes = [None] * NCHUNK
    drained = set()

    def start_read(i):
        r0 = base + i * CHUNK
        reads[i] = pltpu.async_copy(
            w_hbm.at[pl.ds(r0, CHUNK), :], bufs[i % NBUF], rsem[i % NBUF])

    for i in range(min(NBUF, NCHUNK)):
        start_read(i)
    for i in range(NCHUNK):
        reads[i].wait()
        r0 = base + i * CHUNK
        writes[i] = [
            pltpu.async_copy(bufs[i % NBUF],
                             out_hbm.at[pl.ds(b * N + r0, CHUNK), :],
                             wsem[i % NBUF])
            for b in range(B)
        ]
        j = i + NBUF - 1
        if NBUF <= j < NCHUNK:
            for h in writes[j - NBUF]:
                h.wait()
            drained.add(j - NBUF)
            start_read(j)
    for i in range(NCHUNK):
        if i not in drained:
            for h in writes[i]:
                h.wait()


def kernel(x, embed_weight):
    b, n = x.shape
    out = _bcast_copy(embed_weight)
    return out.reshape(b, n, D)
